# baseline (speedup 1.0000x reference)
import functools

import jax
import jax.numpy as jnp
from jax import lax
from jax.experimental import pallas as pl
from jax.experimental.pallas import tpu as pltpu
from jax.experimental.pallas import tpu_sc as plsc

N = 10000
E = 320000
D = 128
G = 64
C = 16

NC = 2
NS = 16
NW = NC * NS
CHUNK = 64
NCHUNK = -(-E // (NW * CHUNK))
NCHUNK += (-NCHUNK) % 4
HLF = NCHUNK // 2
EPT = NCHUNK * CHUNK
EPAD = EPT * NW
NACC = 10240
RPT = NACC // NS

RB = 400
NB = N // RB

_mesh = plsc.VectorSubcoreMesh(
    core_axis_name="c", subcore_axis_name="s", num_cores=NC, num_subcores=NS)


@functools.partial(
    pl.kernel,
    out_type=jax.ShapeDtypeStruct((NC, NACC, D), jnp.float32),
    mesh=_mesh,
    scratch_types=[
        pltpu.VMEM((HLF, CHUNK), jnp.int32),
        pltpu.VMEM((HLF, CHUNK), jnp.int32),
        pltpu.VMEM((2, CHUNK, D), jnp.float32),
        pltpu.VMEM_SHARED((NACC, D), jnp.float32),
        pltpu.SemaphoreType.DMA((2,)),
    ],
)
def _spmm(y_hbm, src_hbm, dst_hbm, zeros_hbm, out_hbm,
          srcidx, dstidx, rows, acc, gsem):
    cid = lax.axis_index("c")
    sid = lax.axis_index("s")
    wid = cid * NS + sid

    pltpu.sync_copy(zeros_hbm, acc.at[pl.ds(sid * RPT, RPT)])

    for h in range(2):
        pltpu.sync_copy(src_hbm.at[wid, pl.ds(h * HLF, HLF)], srcidx)
        pltpu.sync_copy(dst_hbm.at[wid, pl.ds(h * HLF, HLF)], dstidx)
        if h == 0:
            plsc.subcore_barrier()

        pltpu.async_copy(y_hbm.at[srcidx.at[0]], rows.at[0], gsem.at[0])

        def pair(t, carry):
            for b in range(2):
                j = 2 * t + b
                pltpu.make_async_copy(y_hbm.at[srcidx.at[j]], rows.at[b],
                                      gsem.at[b]).wait()
                pltpu.async_copy(y_hbm.at[srcidx.at[j + 1]], rows.at[1 - b],
                                 gsem.at[1 - b])
            return carry

        lax.fori_loop(0, HLF // 2 - 1, pair, 0)
        j = HLF - 2
        pltpu.make_async_copy(y_hbm.at[srcidx.at[j]], rows.at[0],
                              gsem.at[0]).wait()
        pltpu.async_copy(y_hbm.at[srcidx.at[j + 1]], rows.at[1], gsem.at[1])
        pltpu.sync_copy(rows.at[0], acc.at[dstidx.at[j]], add=True)
        pltpu.make_async_copy(y_hbm.at[srcidx.at[j + 1]], rows.at[1],
                              gsem.at[1]).wait()
        pltpu.sync_copy(rows.at[1], acc.at[dstidx.at[j + 1]], add=True)

    plsc.subcore_barrier()
    pltpu.sync_copy(acc.at[pl.ds(sid * RPT, RPT)],
                    out_hbm.at[cid, pl.ds(sid * RPT, RPT)])


def _mm1_body(x_ref, w_ref, o_ref):
    o_ref[...] = jnp.dot(x_ref[...], w_ref[...],
                         preferred_element_type=jnp.float32)


def _mid_body(a0_ref, a1_ref, y_ref, b_ref, w_ref, o_ref):
    s = a0_ref[0] + a1_ref[0] + y_ref[...] + b_ref[...]
    z = jnp.maximum(s, 0.0)
    o_ref[...] = jnp.dot(z, w_ref[...], preferred_element_type=jnp.float32)


def _post_body(a0_ref, a1_ref, y_ref, idx_ref, b2_ref, w3_ref, b3_ref,
               w4_ref, b4_ref, o_ref, pooled_ref):
    i = pl.program_id(0)
    x2 = a0_ref[0] + a1_ref[0] + y_ref[...] + b2_ref[...]
    ids = idx_ref[0]
    gid = lax.broadcasted_iota(jnp.int32, (G, RB), 0)
    onehot = (gid == ids).astype(jnp.float32)
    part = jnp.dot(onehot, x2, preferred_element_type=jnp.float32)

    @pl.when(i == 0)
    def _():
        pooled_ref[...] = jnp.zeros_like(pooled_ref)

    pooled_ref[...] += part

    @pl.when(i == NB - 1)
    def _():
        zg = jnp.maximum(
            jnp.dot(pooled_ref[...], w3_ref[...],
                    preferred_element_type=jnp.float32) + b3_ref[...], 0.0)
        o = jnp.dot(zg, w4_ref[...],
                    preferred_element_type=jnp.float32) + b4_ref[...]
        m = jnp.max(o, axis=1, keepdims=True)
        e = jnp.exp(o - m)
        lse = jnp.log(jnp.sum(e, axis=1, keepdims=True)) + m
        o_ref[...] = o - lse


_full = lambda shape: pl.BlockSpec(shape, lambda i: tuple(0 for _ in shape))

_mm1 = pl.pallas_call(
    _mm1_body,
    grid=(NB,),
    in_specs=[pl.BlockSpec((RB, D), lambda i: (i, 0)), _full((D, D))],
    out_specs=pl.BlockSpec((RB, D), lambda i: (i, 0)),
    out_shape=jax.ShapeDtypeStruct((N, D), jnp.float32),
)

_acc0_spec = pl.BlockSpec((1, RB, D), lambda i: (0, i, 0))
_acc1_spec = pl.BlockSpec((1, RB, D), lambda i: (1, i, 0))

_mid = pl.pallas_call(
    _mid_body,
    grid=(NB,),
    in_specs=[
        _acc0_spec,
        _acc1_spec,
        pl.BlockSpec((RB, D), lambda i: (i, 0)),
        _full((1, D)),
        _full((D, D)),
    ],
    out_specs=pl.BlockSpec((RB, D), lambda i: (i, 0)),
    out_shape=jax.ShapeDtypeStruct((N, D), jnp.float32),
)

_post = pl.pallas_call(
    _post_body,
    grid=(NB,),
    in_specs=[
        _acc0_spec,
        _acc1_spec,
        pl.BlockSpec((RB, D), lambda i: (i, 0)),
        pl.BlockSpec((1, 1, RB), lambda i: (i, 0, 0)),
        _full((1, D)), _full((D, D)), _full((1, D)),
        _full((D, C)), _full((1, C)),
    ],
    out_specs=_full((G, C)),
    out_shape=jax.ShapeDtypeStruct((G, C), jnp.float32),
    scratch_shapes=[pltpu.VMEM((G, D), jnp.float32)],
)


def kernel(x_in, edge_index, idx, W1, b1, W2, b2, W3, b3, W4, b4):
    src = edge_index[0]
    dst = edge_index[1]
    padt = EPT - E // NW
    pad_src = jnp.broadcast_to(
        jnp.arange(padt, dtype=jnp.int32)[None], (NW, padt))
    pad_dst = jnp.broadcast_to(
        (N + jnp.arange(padt, dtype=jnp.int32))[None], (NW, padt))
    srcp = jnp.concatenate(
        [src.reshape(NW, E // NW), pad_src], axis=1).reshape(NW, NCHUNK, CHUNK)
    dstp = jnp.concatenate(
        [dst.reshape(NW, E // NW), pad_dst], axis=1).reshape(NW, NCHUNK, CHUNK)
    zeros = jnp.zeros((RPT, D), jnp.float32)
    idx3 = idx.reshape(NB, 1, RB)
    b1r = b1.reshape(1, D)
    b2r = b2.reshape(1, D)
    b3r = b3.reshape(1, D)
    b4r = b4.reshape(1, C)

    y1 = _mm1(x_in, W1)
    acc1 = _spmm(y1, srcp, dstp, zeros)
    y2 = _mid(acc1, acc1, y1, b1r, W2)
    acc2 = _spmm(y2, srcp, dstp, zeros)
    return _post(acc2, acc2, y2, idx3, b2r, W3, b3r, W4, b4r)


# speedup vs baseline: 1.5647x; 1.5647x over previous
import functools

import jax
import jax.numpy as jnp
from jax import lax
from jax.experimental import pallas as pl
from jax.experimental.pallas import tpu as pltpu
from jax.experimental.pallas import tpu_sc as plsc

N = 10000
E = 320000
D = 128
G = 64
C = 16

NC = 2
NS = 16
NW = NC * NS
CHUNK = 96
NQ = 27
NCHUNK = 4 * NQ
EPT = NCHUNK * CHUNK
NACC = 10112
RPT = NACC // NS

RB = 2000
NB = N // RB

_mesh = plsc.VectorSubcoreMesh(
    core_axis_name="c", subcore_axis_name="s", num_cores=NC, num_subcores=NS)


@functools.partial(
    pl.kernel,
    out_type=jax.ShapeDtypeStruct((NC, NACC, D), jnp.float32),
    mesh=_mesh,
    scratch_types=[
        pltpu.VMEM((NQ, CHUNK), jnp.int32),
        pltpu.VMEM((NQ, CHUNK), jnp.int32),
        pltpu.VMEM((3, CHUNK, D), jnp.float32),
        pltpu.VMEM_SHARED((NACC, D), jnp.float32),
        pltpu.SemaphoreType.DMA((3,)),
        pltpu.SemaphoreType.DMA((3,)),
    ],
)
def _spmm(y_hbm, src_hbm, dst_hbm, zeros_hbm, out_hbm,
          srcidx, dstidx, rows, acc, gsem, ssem):
    cid = lax.axis_index("c")
    sid = lax.axis_index("s")
    wid = cid * NS + sid

    pltpu.sync_copy(zeros_hbm, acc.at[pl.ds(sid * RPT, RPT)])

    def start_g(j, b):
        pltpu.async_copy(y_hbm.at[srcidx.at[j]], rows.at[b], gsem.at[b])

    def wait_g(j, b):
        pltpu.make_async_copy(y_hbm.at[srcidx.at[j]], rows.at[b],
                              gsem.at[b]).wait()

    def start_s(j, b):
        pltpu.async_copy(rows.at[b], acc.at[dstidx.at[j]], ssem.at[b],
                         add=True)

    def wait_s(j, b):
        pltpu.make_async_copy(rows.at[b], acc.at[dstidx.at[j]],
                              ssem.at[b]).wait()

    for h in range(4):
        pltpu.sync_copy(src_hbm.at[wid * 4 + h], srcidx)
        pltpu.sync_copy(dst_hbm.at[wid * 4 + h], dstidx)
        if h == 0:
            plsc.subcore_barrier()

        start_g(0, 0)
        start_g(1, 1)
        wait_g(0, 0)
        start_s(0, 0)
        start_g(2, 2)

        def tri(t, carry):
            for k in range(3):
                j = 1 + 3 * t + k
                b = (1 + k) % 3
                wait_g(j, b)
                start_s(j, b)
                wait_s(j - 1, (b + 2) % 3)
                start_g(j + 2, (b + 2) % 3)
            return carry

        lax.fori_loop(0, (NQ - 3) // 3, tri, 0)
        j = NQ - 2
        wait_g(j, 1)
        start_s(j, 1)
        wait_s(j - 1, 0)
        wait_g(j + 1, 2)
        start_s(j + 1, 2)
        wait_s(j, 1)
        wait_s(j + 1, 2)

    plsc.subcore_barrier()
    pltpu.sync_copy(acc.at[pl.ds(sid * RPT, RPT)],
                    out_hbm.at[cid, pl.ds(sid * RPT, RPT)])


def _mm1_body(x_ref, w_ref, o_ref):
    o_ref[...] = jnp.dot(x_ref[...], w_ref[...],
                         preferred_element_type=jnp.float32)


def _mid_body(a0_ref, a1_ref, y_ref, b_ref, w_ref, o_ref):
    s = a0_ref[0] + a1_ref[0] + y_ref[...] + b_ref[...]
    z = jnp.maximum(s, 0.0)
    o_ref[...] = jnp.dot(z, w_ref[...], preferred_element_type=jnp.float32)


def _post_body(a0_ref, a1_ref, y_ref, idx_ref, b2_ref, w3_ref, b3_ref,
               w4_ref, b4_ref, o_ref, pooled_ref):
    i = pl.program_id(0)
    x2 = a0_ref[0] + a1_ref[0] + y_ref[...] + b2_ref[...]
    ids = idx_ref[0]
    gid = lax.broadcasted_iota(jnp.int32, (G, RB), 0)
    onehot = (gid == ids).astype(jnp.float32)
    part = jnp.dot(onehot, x2, preferred_element_type=jnp.float32)

    @pl.when(i == 0)
    def _():
        pooled_ref[...] = jnp.zeros_like(pooled_ref)

    pooled_ref[...] += part

    @pl.when(i == NB - 1)
    def _():
        zg = jnp.maximum(
            jnp.dot(pooled_ref[...], w3_ref[...],
                    preferred_element_type=jnp.float32) + b3_ref[...], 0.0)
        o = jnp.dot(zg, w4_ref[...],
                    preferred_element_type=jnp.float32) + b4_ref[...]
        m = jnp.max(o, axis=1, keepdims=True)
        e = jnp.exp(o - m)
        lse = jnp.log(jnp.sum(e, axis=1, keepdims=True)) + m
        o_ref[...] = o - lse


_full = lambda shape: pl.BlockSpec(shape, lambda i: tuple(0 for _ in shape))

_mm1 = pl.pallas_call(
    _mm1_body,
    grid=(NB,),
    in_specs=[pl.BlockSpec((RB, D), lambda i: (i, 0)), _full((D, D))],
    out_specs=pl.BlockSpec((RB, D), lambda i: (i, 0)),
    out_shape=jax.ShapeDtypeStruct((N, D), jnp.float32),
)

_acc0_spec = pl.BlockSpec((1, RB, D), lambda i: (0, i, 0))
_acc1_spec = pl.BlockSpec((1, RB, D), lambda i: (1, i, 0))

_mid = pl.pallas_call(
    _mid_body,
    grid=(NB,),
    in_specs=[
        _acc0_spec,
        _acc1_spec,
        pl.BlockSpec((RB, D), lambda i: (i, 0)),
        _full((1, D)),
        _full((D, D)),
    ],
    out_specs=pl.BlockSpec((RB, D), lambda i: (i, 0)),
    out_shape=jax.ShapeDtypeStruct((N, D), jnp.float32),
)

_post = pl.pallas_call(
    _post_body,
    grid=(NB,),
    in_specs=[
        _acc0_spec,
        _acc1_spec,
        pl.BlockSpec((RB, D), lambda i: (i, 0)),
        pl.BlockSpec((1, 1, RB), lambda i: (i, 0, 0)),
        _full((1, D)), _full((D, D)), _full((1, D)),
        _full((D, C)), _full((1, C)),
    ],
    out_specs=_full((G, C)),
    out_shape=jax.ShapeDtypeStruct((G, C), jnp.float32),
    scratch_shapes=[pltpu.VMEM((G, D), jnp.float32)],
)


def kernel(x_in, edge_index, idx, W1, b1, W2, b2, W3, b3, W4, b4):
    src = edge_index[0]
    dst = edge_index[1]
    padt = EPT - E // NW
    pad_src = jnp.broadcast_to(
        jnp.arange(padt, dtype=jnp.int32)[None], (NW, padt))
    pad_dst = jnp.broadcast_to(
        (N + jnp.arange(padt, dtype=jnp.int32) % (NACC - N))[None], (NW, padt))
    srcp = jnp.concatenate(
        [src.reshape(NW, E // NW), pad_src], axis=1).reshape(NW * 4, NQ, CHUNK)
    dstp = jnp.concatenate(
        [dst.reshape(NW, E // NW), pad_dst], axis=1).reshape(NW * 4, NQ, CHUNK)
    zeros = jnp.zeros((RPT, D), jnp.float32)
    idx3 = idx.reshape(NB, 1, RB)
    b1r = b1.reshape(1, D)
    b2r = b2.reshape(1, D)
    b3r = b3.reshape(1, D)
    b4r = b4.reshape(1, C)

    y1 = _mm1(x_in, W1)
    acc1 = _spmm(y1, srcp, dstp, zeros)
    y2 = _mid(acc1, acc1, y1, b1r, W2)
    acc2 = _spmm(y2, srcp, dstp, zeros)
    return _post(acc2, acc2, y2, idx3, b2r, W3, b3r, W4, b4r)


# speedup vs baseline: 1.7903x; 1.1442x over previous
import functools

import jax
import jax.numpy as jnp
from jax import lax
from jax.experimental import pallas as pl
from jax.experimental.pallas import tpu as pltpu
from jax.experimental.pallas import tpu_sc as plsc

N = 10000
E = 320000
D = 128
G = 64
C = 16

NC = 2
NS = 16
NW = NC * NS
CHUNK = 96
NQ = 18
NP = 6
NCHUNK = NP * NQ
EPT = NCHUNK * CHUNK
NACC = 10112
RPT = NACC // NS
TAIL = N - (NS - 1) * RPT

RB = 2000
NB = N // RB

_mesh = plsc.VectorSubcoreMesh(
    core_axis_name="c", subcore_axis_name="s", num_cores=NC, num_subcores=NS)


@functools.partial(
    pl.kernel,
    out_type=jax.ShapeDtypeStruct((NC, NACC, D), jnp.float32),
    mesh=_mesh,
    scratch_types=[
        pltpu.VMEM((2, NQ, CHUNK), jnp.int32),
        pltpu.VMEM((2, NQ, CHUNK), jnp.int32),
        pltpu.VMEM((3, CHUNK, D), jnp.float32),
        pltpu.VMEM_SHARED((NACC, D), jnp.float32),
        pltpu.SemaphoreType.DMA((3,)),
        pltpu.SemaphoreType.DMA((3,)),
        pltpu.SemaphoreType.DMA((2,)),
    ],
)
def _spmm(y_hbm, src_hbm, dst_hbm, zeros_hbm, out_hbm,
          srcidx, dstidx, rows, acc, gsem, ssem, isem):
    cid = lax.axis_index("c")
    sid = lax.axis_index("s")
    wid = cid * NS + sid

    def start_g(j, b):
        q = (j // NQ) % 2
        r = j % NQ
        pltpu.async_copy(y_hbm.at[srcidx.at[q, r]], rows.at[b], gsem.at[b])

    def wait_g(j, b):
        q = (j // NQ) % 2
        r = j % NQ
        pltpu.make_async_copy(y_hbm.at[srcidx.at[q, r]], rows.at[b],
                              gsem.at[b]).wait()

    def start_s(j, b):
        q = (j // NQ) % 2
        r = j % NQ
        pltpu.async_copy(rows.at[b], acc.at[dstidx.at[q, r]], ssem.at[b],
                         add=True)

    def wait_s(j, b):
        q = (j // NQ) % 2
        r = j % NQ
        pltpu.make_async_copy(rows.at[b], acc.at[dstidx.at[q, r]],
                              ssem.at[b]).wait()

    def start_idx(h, slot):
        pltpu.async_copy(src_hbm.at[wid * NP + h], srcidx.at[slot],
                         isem.at[slot])
        pltpu.async_copy(dst_hbm.at[wid * NP + h], dstidx.at[slot],
                         isem.at[slot])

    def wait_idx(slot):
        pltpu.make_async_copy(src_hbm.at[wid * NP], srcidx.at[slot],
                              isem.at[slot]).wait()
        pltpu.make_async_copy(dst_hbm.at[wid * NP], dstidx.at[slot],
                              isem.at[slot]).wait()

    @pl.when(cid == 0)
    def _():
        @pl.when(sid < NS - 1)
        def _():
            pltpu.sync_copy(y_hbm.at[pl.ds(sid * RPT, RPT)],
                            acc.at[pl.ds(sid * RPT, RPT)])

        @pl.when(sid == NS - 1)
        def _():
            pltpu.sync_copy(y_hbm.at[pl.ds((NS - 1) * RPT, TAIL)],
                            acc.at[pl.ds((NS - 1) * RPT, TAIL)])
            pltpu.sync_copy(zeros_hbm.at[pl.ds(0, NACC - N)],
                            acc.at[pl.ds(N, NACC - N)])

    @pl.when(cid == 1)
    def _():
        pltpu.sync_copy(zeros_hbm, acc.at[pl.ds(sid * RPT, RPT)])

    pltpu.sync_copy(src_hbm.at[wid * NP], srcidx.at[0])
    pltpu.sync_copy(dst_hbm.at[wid * NP], dstidx.at[0])
    start_idx(1, 1)
    start_g(0, 0)
    start_g(1, 1)
    plsc.subcore_barrier()
    wait_g(0, 0)
    start_s(0, 0)
    start_g(2, 2)

    def tri(t, carry):
        for k in range(3):
            j = 1 + 3 * t + k
            b = (1 + k) % 3
            wait_g(j, b)
            start_s(j, b)
            wait_s(j - 1, (b + 2) % 3)
            q = (j // NQ) % 2
            r = j % NQ

            @pl.when(jnp.logical_and(r == 0, j < (NP - 2) * NQ + 1))
            def _():
                start_idx(j // NQ + 1, (q + 1) % 2)

            @pl.when(r == NQ - 2)
            def _():
                wait_idx((q + 1) % 2)

            start_g(j + 2, (b + 2) % 3)
        return carry

    lax.fori_loop(0, (NCHUNK - 3) // 3, tri, 0)
    j = NCHUNK - 2
    wait_g(j, 1)
    start_s(j, 1)
    wait_s(j - 1, 0)
    wait_g(j + 1, 2)
    start_s(j + 1, 2)
    wait_s(j, 1)
    wait_s(j + 1, 2)

    plsc.subcore_barrier()
    pltpu.sync_copy(acc.at[pl.ds(sid * RPT, RPT)],
                    out_hbm.at[cid, pl.ds(sid * RPT, RPT)])


def _mid_body(a0_ref, a1_ref, b1_ref, w1_ref, w2_ref, o_ref):
    s = a0_ref[0] + a1_ref[0]
    z = jnp.maximum(
        jnp.dot(s, w1_ref[...], preferred_element_type=jnp.float32)
        + b1_ref[...], 0.0)
    o_ref[...] = jnp.dot(z, w2_ref[...], preferred_element_type=jnp.float32)


def _post_body(a0_ref, a1_ref, idx_ref, b2_ref, w3_ref, b3_ref,
               w4_ref, b4_ref, o_ref, pooled_ref):
    i = pl.program_id(0)
    x2 = a0_ref[0] + a1_ref[0] + b2_ref[...]
    ids = idx_ref[0]
    gid = lax.broadcasted_iota(jnp.int32, (G, RB), 0)
    onehot = (gid == ids).astype(jnp.float32)
    part = jnp.dot(onehot, x2, preferred_element_type=jnp.float32)

    @pl.when(i == 0)
    def _():
        pooled_ref[...] = jnp.zeros_like(pooled_ref)

    pooled_ref[...] += part

    @pl.when(i == NB - 1)
    def _():
        zg = jnp.maximum(
            jnp.dot(pooled_ref[...], w3_ref[...],
                    preferred_element_type=jnp.float32) + b3_ref[...], 0.0)
        o = jnp.dot(zg, w4_ref[...],
                    preferred_element_type=jnp.float32) + b4_ref[...]
        m = jnp.max(o, axis=1, keepdims=True)
        e = jnp.exp(o - m)
        lse = jnp.log(jnp.sum(e, axis=1, keepdims=True)) + m
        o_ref[...] = o - lse


_full = lambda shape: pl.BlockSpec(shape, lambda i: tuple(0 for _ in shape))

_acc0_spec = pl.BlockSpec((1, RB, D), lambda i: (0, i, 0))
_acc1_spec = pl.BlockSpec((1, RB, D), lambda i: (1, i, 0))

_mid = pl.pallas_call(
    _mid_body,
    grid=(NB,),
    in_specs=[
        _acc0_spec,
        _acc1_spec,
        _full((1, D)),
        _full((D, D)),
        _full((D, D)),
    ],
    out_specs=pl.BlockSpec((RB, D), lambda i: (i, 0)),
    out_shape=jax.ShapeDtypeStruct((N, D), jnp.float32),
)

_post = pl.pallas_call(
    _post_body,
    grid=(NB,),
    in_specs=[
        _acc0_spec,
        _acc1_spec,
        pl.BlockSpec((1, 1, RB), lambda i: (i, 0, 0)),
        _full((1, D)), _full((D, D)), _full((1, D)),
        _full((D, C)), _full((1, C)),
    ],
    out_specs=_full((G, C)),
    out_shape=jax.ShapeDtypeStruct((G, C), jnp.float32),
    scratch_shapes=[pltpu.VMEM((G, D), jnp.float32)],
)


def kernel(x_in, edge_index, idx, W1, b1, W2, b2, W3, b3, W4, b4):
    src = edge_index[0]
    dst = edge_index[1]
    padt = EPT - E // NW
    pad_src = jnp.broadcast_to(
        jnp.arange(padt, dtype=jnp.int32)[None], (NW, padt))
    pad_dst = jnp.broadcast_to(
        (N + jnp.arange(padt, dtype=jnp.int32) % (NACC - N))[None], (NW, padt))
    srcp = jnp.concatenate(
        [src.reshape(NW, E // NW), pad_src], axis=1).reshape(NW * NP, NQ, CHUNK)
    dstp = jnp.concatenate(
        [dst.reshape(NW, E // NW), pad_dst], axis=1).reshape(NW * NP, NQ, CHUNK)
    zeros = jnp.zeros((RPT, D), jnp.float32)
    idx3 = idx.reshape(NB, 1, RB)
    b1r = b1.reshape(1, D)
    b2r = b2.reshape(1, D)
    b3r = b3.reshape(1, D)
    b4r = b4.reshape(1, C)

    acc1 = _spmm(x_in, srcp, dstp, zeros)
    y2 = _mid(acc1, acc1, b1r, W1, W2)
    acc2 = _spmm(y2, srcp, dstp, zeros)
    return _post(acc2, acc2, idx3, b2r, W3, b3r, W4, b4r)


# speedup vs baseline: 1.8394x; 1.0274x over previous
import functools

import jax
import jax.numpy as jnp
from jax import lax
from jax.experimental import pallas as pl
from jax.experimental.pallas import tpu as pltpu
from jax.experimental.pallas import tpu_sc as plsc

N = 10000
E = 320000
D = 128
G = 64
C = 16

NC = 2
NS = 16
NW = NC * NS
CHUNK = 96
NQ = 21
NP = 5
NCHUNK = NP * NQ
EPT = NCHUNK * CHUNK
NACC = 10112
RPT = NACC // NS
TAIL = N - (NS - 1) * RPT

RB = 2000
NB = N // RB

_mesh = plsc.VectorSubcoreMesh(
    core_axis_name="c", subcore_axis_name="s", num_cores=NC, num_subcores=NS)


@functools.partial(
    pl.kernel,
    out_type=jax.ShapeDtypeStruct((NC, NACC, D), jnp.float32),
    mesh=_mesh,
    scratch_types=[
        pltpu.VMEM((2, NQ, CHUNK), jnp.int32),
        pltpu.VMEM((2, NQ, CHUNK), jnp.int32),
        pltpu.VMEM((3, CHUNK, D), jnp.float32),
        pltpu.VMEM_SHARED((NACC, D), jnp.float32),
        pltpu.SemaphoreType.DMA((3,)),
        pltpu.SemaphoreType.DMA((3,)),
        pltpu.SemaphoreType.DMA((2,)),
        pltpu.SemaphoreType.DMA,
    ],
)
def _spmm(y_hbm, src_hbm, dst_hbm, zeros_hbm, out_hbm,
          srcidx, dstidx, rows, acc, gsem, ssem, isem, vsem):
    cid = lax.axis_index("c")
    sid = lax.axis_index("s")
    wid = cid * NS + sid

    def start_g(j, b):
        q = (j // NQ) % 2
        r = j % NQ
        pltpu.async_copy(y_hbm.at[srcidx.at[q, r]], rows.at[b], gsem.at[b])

    def wait_g(j, b):
        q = (j // NQ) % 2
        r = j % NQ
        pltpu.make_async_copy(y_hbm.at[srcidx.at[q, r]], rows.at[b],
                              gsem.at[b]).wait()

    def start_s(j, b):
        q = (j // NQ) % 2
        r = j % NQ
        pltpu.async_copy(rows.at[b], acc.at[dstidx.at[q, r]], ssem.at[b],
                         add=True)

    def wait_s(j, b):
        q = (j // NQ) % 2
        r = j % NQ
        pltpu.make_async_copy(rows.at[b], acc.at[dstidx.at[q, r]],
                              ssem.at[b]).wait()

    def start_idx(h, slot):
        pltpu.async_copy(src_hbm.at[wid * NP + h], srcidx.at[slot],
                         isem.at[slot])
        pltpu.async_copy(dst_hbm.at[wid * NP + h], dstidx.at[slot],
                         isem.at[slot])

    def wait_idx(slot):
        pltpu.make_async_copy(src_hbm.at[wid * NP], srcidx.at[slot],
                              isem.at[slot]).wait()
        pltpu.make_async_copy(dst_hbm.at[wid * NP], dstidx.at[slot],
                              isem.at[slot]).wait()

    @pl.when(cid == 0)
    def _():
        @pl.when(sid < NS - 1)
        def _():
            pltpu.async_copy(y_hbm.at[pl.ds(sid * RPT, RPT)],
                             acc.at[pl.ds(sid * RPT, RPT)], vsem)

        @pl.when(sid == NS - 1)
        def _():
            pltpu.async_copy(y_hbm.at[pl.ds((NS - 1) * RPT, TAIL)],
                             acc.at[pl.ds((NS - 1) * RPT, TAIL)], vsem)
            pltpu.async_copy(zeros_hbm.at[pl.ds(0, NACC - N)],
                             acc.at[pl.ds(N, NACC - N)], vsem)

    @pl.when(cid == 1)
    def _():
        pltpu.async_copy(zeros_hbm, acc.at[pl.ds(sid * RPT, RPT)], vsem)

    pltpu.sync_copy(src_hbm.at[wid * NP], srcidx.at[0])
    pltpu.sync_copy(dst_hbm.at[wid * NP], dstidx.at[0])
    start_idx(1, 1)
    start_g(0, 0)
    start_g(1, 1)

    @pl.when(cid == 0)
    def _():
        @pl.when(sid < NS - 1)
        def _():
            pltpu.make_async_copy(y_hbm.at[pl.ds(sid * RPT, RPT)],
                                  acc.at[pl.ds(sid * RPT, RPT)], vsem).wait()

        @pl.when(sid == NS - 1)
        def _():
            pltpu.make_async_copy(y_hbm.at[pl.ds((NS - 1) * RPT, TAIL)],
                                  acc.at[pl.ds((NS - 1) * RPT, TAIL)],
                                  vsem).wait()
            pltpu.make_async_copy(zeros_hbm.at[pl.ds(0, NACC - N)],
                                  acc.at[pl.ds(N, NACC - N)], vsem).wait()

    @pl.when(cid == 1)
    def _():
        pltpu.make_async_copy(zeros_hbm, acc.at[pl.ds(sid * RPT, RPT)],
                              vsem).wait()

    plsc.subcore_barrier()
    wait_g(0, 0)
    start_s(0, 0)
    start_g(2, 2)

    def tri(t, carry):
        for k in range(3):
            j = 1 + 3 * t + k
            b = (1 + k) % 3
            wait_g(j, b)
            start_s(j, b)
            wait_s(j - 1, (b + 2) % 3)
            q = (j // NQ) % 2
            r = j % NQ

            @pl.when(jnp.logical_and(r == 0, j < (NP - 2) * NQ + 1))
            def _():
                start_idx(j // NQ + 1, (q + 1) % 2)

            @pl.when(r == NQ - 2)
            def _():
                wait_idx((q + 1) % 2)

            start_g(j + 2, (b + 2) % 3)
        return carry

    lax.fori_loop(0, (NCHUNK - 3) // 3, tri, 0)
    j = NCHUNK - 2
    wait_g(j, 1)
    start_s(j, 1)
    wait_s(j - 1, 0)
    wait_g(j + 1, 2)
    start_s(j + 1, 2)
    wait_s(j, 1)
    wait_s(j + 1, 2)

    plsc.subcore_barrier()
    pltpu.sync_copy(acc.at[pl.ds(sid * RPT, RPT)],
                    out_hbm.at[cid, pl.ds(sid * RPT, RPT)])


def _mid_body(a0_ref, a1_ref, b1_ref, w1_ref, w2_ref, o_ref):
    s = a0_ref[0] + a1_ref[0]
    z = jnp.maximum(
        jnp.dot(s, w1_ref[...], preferred_element_type=jnp.float32)
        + b1_ref[...], 0.0)
    o_ref[...] = jnp.dot(z, w2_ref[...], preferred_element_type=jnp.float32)


def _post_body(a0_ref, a1_ref, idx_ref, b2_ref, w3_ref, b3_ref,
               w4_ref, b4_ref, o_ref, pooled_ref):
    i = pl.program_id(0)
    x2 = a0_ref[0] + a1_ref[0] + b2_ref[...]
    ids = idx_ref[0]
    gid = lax.broadcasted_iota(jnp.int32, (G, RB), 0)
    onehot = (gid == ids).astype(jnp.float32)
    part = jnp.dot(onehot, x2, preferred_element_type=jnp.float32)

    @pl.when(i == 0)
    def _():
        pooled_ref[...] = jnp.zeros_like(pooled_ref)

    pooled_ref[...] += part

    @pl.when(i == NB - 1)
    def _():
        zg = jnp.maximum(
            jnp.dot(pooled_ref[...], w3_ref[...],
                    preferred_element_type=jnp.float32) + b3_ref[...], 0.0)
        o = jnp.dot(zg, w4_ref[...],
                    preferred_element_type=jnp.float32) + b4_ref[...]
        m = jnp.max(o, axis=1, keepdims=True)
        e = jnp.exp(o - m)
        lse = jnp.log(jnp.sum(e, axis=1, keepdims=True)) + m
        o_ref[...] = o - lse


_full = lambda shape: pl.BlockSpec(shape, lambda i: tuple(0 for _ in shape))

_acc0_spec = pl.BlockSpec((1, RB, D), lambda i: (0, i, 0))
_acc1_spec = pl.BlockSpec((1, RB, D), lambda i: (1, i, 0))

_mid = pl.pallas_call(
    _mid_body,
    grid=(NB,),
    in_specs=[
        _acc0_spec,
        _acc1_spec,
        _full((1, D)),
        _full((D, D)),
        _full((D, D)),
    ],
    out_specs=pl.BlockSpec((RB, D), lambda i: (i, 0)),
    out_shape=jax.ShapeDtypeStruct((N, D), jnp.float32),
)

_post = pl.pallas_call(
    _post_body,
    grid=(NB,),
    in_specs=[
        _acc0_spec,
        _acc1_spec,
        pl.BlockSpec((1, 1, RB), lambda i: (i, 0, 0)),
        _full((1, D)), _full((D, D)), _full((1, D)),
        _full((D, C)), _full((1, C)),
    ],
    out_specs=_full((G, C)),
    out_shape=jax.ShapeDtypeStruct((G, C), jnp.float32),
    scratch_shapes=[pltpu.VMEM((G, D), jnp.float32)],
)


def kernel(x_in, edge_index, idx, W1, b1, W2, b2, W3, b3, W4, b4):
    src = edge_index[0]
    dst = edge_index[1]
    padt = EPT - E // NW
    pad_src = jnp.broadcast_to(
        jnp.arange(padt, dtype=jnp.int32)[None], (NW, padt))
    pad_dst = jnp.broadcast_to(
        (N + jnp.arange(padt, dtype=jnp.int32) % (NACC - N))[None], (NW, padt))
    srcp = jnp.concatenate(
        [src.reshape(NW, E // NW), pad_src], axis=1).reshape(NW * NP, NQ, CHUNK)
    dstp = jnp.concatenate(
        [dst.reshape(NW, E // NW), pad_dst], axis=1).reshape(NW * NP, NQ, CHUNK)
    zeros = jnp.zeros((RPT, D), jnp.float32)
    idx3 = idx.reshape(NB, 1, RB)
    b1r = b1.reshape(1, D)
    b2r = b2.reshape(1, D)
    b3r = b3.reshape(1, D)
    b4r = b4.reshape(1, C)

    acc1 = _spmm(x_in, srcp, dstp, zeros)
    y2 = _mid(acc1, acc1, b1r, W1, W2)
    acc2 = _spmm(y2, srcp, dstp, zeros)
    return _post(acc2, acc2, idx3, b2r, W3, b3r, W4, b4r)
